# baseline (speedup 1.0000x reference)
import functools

import jax
import jax.numpy as jnp
from jax import lax
from jax.experimental import pallas as pl
from jax.experimental.pallas import tpu as pltpu
from jax.experimental.pallas import tpu_sc as plsc

_NC = 2
_NS = 16
_NW = _NC * _NS

_ROWS_PER_CHUNK = 4
_NBUF = 2


@functools.partial(jax.jit, static_argnames=("dims",))
def _sc_lookup(W, x, *, dims):
    b0, b1, dim = dims
    w0 = (b1 // 2 + 7) // 8 * 8
    windows = ((0, w0), (w0, b1 - w0))
    rows_per_w = b0 // _NW
    chunks = rows_per_w // _ROWS_PER_CHUNK
    assert chunks % _NBUF == 0

    mesh = plsc.VectorSubcoreMesh(core_axis_name="c", subcore_axis_name="s")

    @functools.partial(
        pl.kernel,
        mesh=mesh,
        compiler_params=pltpu.CompilerParams(use_tc_tiling_on_sc=False),
        out_type=jax.ShapeDtypeStruct((b0, b1, dim), jnp.float32),
        scratch_types=[
            pltpu.VMEM((_NBUF, _ROWS_PER_CHUNK, b1), jnp.int32),
            pltpu.VMEM((_NBUF, _ROWS_PER_CHUNK, b1, dim), jnp.float32),
            pltpu.SemaphoreType.DMA((_NBUF,)),
            pltpu.SemaphoreType.DMA((_NBUF,)),
            pltpu.SemaphoreType.DMA((_NBUF,)),
        ],
    )
    def body(table_hbm, x_hbm, out3d, idx_v, rows_v, isem, gsem, ssem):
        wid = lax.axis_index("s") * _NC + lax.axis_index("c")
        row0 = wid * rows_per_w

        def idx_load(g, b):
            return pltpu.make_async_copy(
                x_hbm.at[pl.ds(row0 + g * _ROWS_PER_CHUNK, _ROWS_PER_CHUNK)],
                idx_v.at[b],
                isem.at[b],
            )

        def gather(g, b, j):
            r, h = divmod(j, 2)
            off, w = windows[h]
            return pltpu.make_async_copy(
                table_hbm.at[idx_v.at[b, r, pl.ds(off, w)]],
                rows_v.at[b, r, pl.ds(off, w)],
                gsem.at[b],
            )

        def store(g, b):
            return pltpu.make_async_copy(
                rows_v.at[b],
                out3d.at[pl.ds(row0 + g * _ROWS_PER_CHUNK, _ROWS_PER_CHUNK)],
                ssem.at[b],
            )

        for b in range(_NBUF):
            idx_load(b, b).start()

        def outer(i, carry):
            g0 = i * _NBUF
            for b in range(_NBUF):
                g = g0 + b
                idx_load(g, b).wait()
                @pl.when(g0 > 0)
                def _():
                    store(g, b).wait()

                for j in range(2 * _ROWS_PER_CHUNK):
                    gather(g, b, j).start()
                for j in range(2 * _ROWS_PER_CHUNK):
                    gather(g, b, j).wait()

                @pl.when(g0 < chunks - _NBUF)
                def _():
                    idx_load(g + _NBUF, b).start()

                store(g, b).start()
            return carry

        lax.fori_loop(0, chunks // _NBUF, outer, 0)
        for b in range(_NBUF):
            store(chunks - _NBUF + b, b).wait()

    return body(W, x)


def kernel(x, W):
    b0, b1 = x.shape
    dim = W.shape[1]
    return _sc_lookup(W, x.astype(jnp.int32), dims=(b0, b1, dim))


# speedup vs baseline: 1.8886x; 1.8886x over previous
import functools

import jax
import jax.numpy as jnp
from jax import lax
from jax.experimental import pallas as pl
from jax.experimental.pallas import tpu as pltpu
from jax.experimental.pallas import tpu_sc as plsc

_NC = 2
_NS = 16
_NW = _NC * _NS

_IDX_W = 128
_ROWS_PER_CHUNK = 2
_CHUNK = _IDX_W * _ROWS_PER_CHUNK
_NBUF = 2

_RL_ROWS = 32


@functools.partial(jax.jit, static_argnames=("n2", "dim2"))
def _sc_lookup(W2, idx2d, *, n2, dim2):
    per_w = n2 // _NW
    chunks = per_w // _CHUNK
    rows_per_w = per_w // _IDX_W
    assert chunks % _NBUF == 0

    mesh = plsc.VectorSubcoreMesh(core_axis_name="c", subcore_axis_name="s")

    @functools.partial(
        pl.kernel,
        mesh=mesh,
        out_type=jax.ShapeDtypeStruct((n2, dim2), jnp.float32),
        scratch_types=[
            pltpu.VMEM((_NBUF, _ROWS_PER_CHUNK, _IDX_W), jnp.int32),
            pltpu.VMEM((_NBUF, _CHUNK, dim2), jnp.float32),
            pltpu.SemaphoreType.DMA((_NBUF,)),
            pltpu.SemaphoreType.DMA((_NBUF,)),
            pltpu.SemaphoreType.DMA((_NBUF,)),
        ],
    )
    def body(table_hbm, idx_hbm, out_hbm, idx_v, rows_v, isem, gsem, ssem):
        wid = lax.axis_index("s") * _NC + lax.axis_index("c")
        row0 = wid * rows_per_w

        def idx_load(g, b):
            return pltpu.make_async_copy(
                idx_hbm.at[pl.ds(row0 + g * _ROWS_PER_CHUNK, _ROWS_PER_CHUNK)],
                idx_v.at[b],
                isem.at[b],
            )

        def gather(g, b, j):
            return pltpu.make_async_copy(
                table_hbm.at[idx_v.at[b, j]],
                rows_v.at[b, pl.ds(j * _IDX_W, _IDX_W)],
                gsem.at[b],
            )

        def store(g, b):
            return pltpu.make_async_copy(
                rows_v.at[b],
                out_hbm.at[pl.ds((row0 + g * _ROWS_PER_CHUNK) * _IDX_W, _CHUNK)],
                ssem.at[b],
            )

        for b in range(_NBUF):
            idx_load(b, b).start()

        def outer(i, carry):
            g0 = i * _NBUF
            for b in range(_NBUF):
                g = g0 + b
                idx_load(g, b).wait()
                @pl.when(g0 > 0)
                def _():
                    store(g, b).wait()

                for j in range(_ROWS_PER_CHUNK):
                    gather(g, b, j).start()
                for j in range(_ROWS_PER_CHUNK):
                    gather(g, b, j).wait()

                @pl.when(g0 < chunks - _NBUF)
                def _():
                    idx_load(g + _NBUF, b).start()

                store(g, b).start()
            return carry

        lax.fori_loop(0, chunks // _NBUF, outer, 0)
        for b in range(_NBUF):
            store(chunks - _NBUF + b, b).wait()

    return body(W2, idx2d)


def _relayout_body(in_ref, out_ref):
    a = in_ref[...]
    dim = out_ref.shape[-1]
    l = a[:, :dim]
    r = a[:, dim:]
    lr = jnp.stack([l, r], axis=1)
    out_ref[...] = lr.reshape(out_ref.shape)


@functools.partial(jax.jit, static_argnames=("dims",))
def _tc_relayout(flat, *, dims):
    b0, b1, dim = dims
    rows_in = _RL_ROWS * b1 // 2
    return pl.pallas_call(
        _relayout_body,
        grid=(b0 // _RL_ROWS,),
        in_specs=[
            pl.BlockSpec((rows_in, 2 * dim), lambda i: (i, 0)),
        ],
        out_specs=pl.BlockSpec((_RL_ROWS, b1, dim), lambda i: (i, 0, 0)),
        out_shape=jax.ShapeDtypeStruct((b0, b1, dim), jnp.float32),
    )(flat)


def kernel(x, W):
    b0, b1 = x.shape
    vocab, dim = W.shape
    n2 = b0 * b1 // 2
    W2 = jnp.concatenate(
        [jnp.repeat(W, vocab, axis=0), jnp.tile(W, (vocab, 1))], axis=1
    )
    xp = x.reshape(n2, 2).astype(jnp.int32)
    idx2 = xp[:, 0] * vocab + xp[:, 1]
    idx2d = idx2.reshape(n2 // _IDX_W, _IDX_W)
    flat = _sc_lookup(W2, idx2d, n2=n2, dim2=2 * dim)
    return _tc_relayout(flat, dims=(b0, b1, dim))


# speedup vs baseline: 2.9204x; 1.5463x over previous
import functools

import jax
import jax.numpy as jnp
from jax import lax
from jax.experimental import pallas as pl
from jax.experimental.pallas import tpu as pltpu
from jax.experimental.pallas import tpu_sc as plsc

_NC = 2
_NS = 16
_NW = _NC * _NS

_IDX_W = 128
_ROWS_PER_CHUNK = 2
_CHUNK = _IDX_W * _ROWS_PER_CHUNK
_NBUF = 2

_RL_ROWS = 32


@functools.partial(jax.jit, static_argnames=("n2", "dim2"))
def _sc_lookup(W2, idx2d, *, n2, dim2):
    per_w = n2 // _NW
    chunks = per_w // _CHUNK
    rows_per_w = per_w // _IDX_W
    assert chunks % _NBUF == 0

    mesh = plsc.VectorSubcoreMesh(core_axis_name="c", subcore_axis_name="s")

    @functools.partial(
        pl.kernel,
        mesh=mesh,
        out_type=jax.ShapeDtypeStruct((n2, dim2), jnp.float32),
        scratch_types=[
            pltpu.VMEM((_NBUF, _ROWS_PER_CHUNK, _IDX_W), jnp.int32),
            pltpu.VMEM((_NBUF, _CHUNK, dim2), jnp.float32),
            pltpu.SemaphoreType.DMA((_NBUF,)),
            pltpu.SemaphoreType.DMA((_NBUF,)),
            pltpu.SemaphoreType.DMA((_NBUF,)),
        ],
    )
    def body(table_hbm, idx_hbm, out_hbm, idx_v, rows_v, isem, gsem, ssem):
        wid = lax.axis_index("s") * _NC + lax.axis_index("c")
        row0 = wid * rows_per_w

        def idx_load(g, b):
            return pltpu.make_async_copy(
                idx_hbm.at[pl.ds(row0 + g * _ROWS_PER_CHUNK, _ROWS_PER_CHUNK)],
                idx_v.at[b],
                isem.at[b],
            )

        def gather(g, b, j):
            return pltpu.make_async_copy(
                table_hbm.at[idx_v.at[b, j]],
                rows_v.at[b, pl.ds(j * _IDX_W, _IDX_W)],
                gsem.at[b],
            )

        def store(g, b):
            return pltpu.make_async_copy(
                rows_v.at[b],
                out_hbm.at[pl.ds((row0 + g * _ROWS_PER_CHUNK) * _IDX_W, _CHUNK)],
                ssem.at[b],
            )

        for b in range(_NBUF):
            idx_load(b, b).start()

        def outer(i, carry):
            g0 = i * _NBUF
            for b in range(_NBUF):
                g = g0 + b
                idx_load(g, b).wait()
                @pl.when(g0 > 0)
                def _():
                    store(g, b).wait()

                for j in range(_ROWS_PER_CHUNK):
                    gather(g, b, j).start()
                for j in range(_ROWS_PER_CHUNK):
                    gather(g, b, j).wait()

                @pl.when(g0 < chunks - _NBUF)
                def _():
                    idx_load(g + _NBUF, b).start()

                store(g, b).start()
            return carry

        lax.fori_loop(0, chunks // _NBUF, outer, 0)
        for b in range(_NBUF):
            store(chunks - _NBUF + b, b).wait()

    return body(W2, idx2d)


def _relayout_body(in_ref, out_ref):
    a = in_ref[...]
    dim = out_ref.shape[-1]
    l = a[:, :dim]
    r = a[:, dim:]
    lr = jnp.stack([l, r], axis=1)
    out_ref[...] = lr.reshape(out_ref.shape)


@functools.partial(jax.jit, static_argnames=("dims",))
def _tc_relayout(flat, *, dims):
    b0, b1, dim = dims
    rows_in = _RL_ROWS * b1 // 2
    return pl.pallas_call(
        _relayout_body,
        grid=(b0 // _RL_ROWS,),
        in_specs=[
            pl.BlockSpec((rows_in, 2 * dim), lambda i: (i, 0)),
        ],
        out_specs=pl.BlockSpec((_RL_ROWS, b1, dim), lambda i: (i, 0, 0)),
        out_shape=jax.ShapeDtypeStruct((b0, b1, dim), jnp.float32),
    )(flat)


def kernel(x, W):
    b0, b1 = x.shape
    vocab, dim = W.shape
    n2 = b0 * b1 // 2
    W2 = jnp.concatenate(
        [jnp.repeat(W, vocab, axis=0), jnp.tile(W, (vocab, 1))], axis=1
    )
    xi = x.astype(jnp.int32)
    idx2 = xi[:, 0::2] * vocab + xi[:, 1::2]
    idx2d = idx2.reshape(n2 // _IDX_W, _IDX_W)
    flat = _sc_lookup(W2, idx2d, n2=n2, dim2=2 * dim)
    return flat.reshape(b0, b1, dim)
